# baseline (speedup 1.0000x reference)
import functools

import jax
import jax.numpy as jnp
from jax import lax
from jax.experimental import pallas as pl
from jax.experimental.pallas import tpu as pltpu
from jax.experimental.pallas import tpu_sc as plsc


def kernel(inputs, word_table, pos_table):
    B, S = inputs.shape
    V, D = word_table.shape
    info = plsc.get_sparse_core_info()
    NC, NS, L = info.num_cores, info.num_subcores, info.num_lanes
    NW = NC * NS
    assert B % NW == 0 and D % L == 0 and S % 8 == 0 and V % 2 == 0
    seqs_per_w = B // NW
    SP = ((S + L - 1) // L) * L
    NG = SP // L
    JD = D // L

    idx_flat = inputs.reshape(B * S)
    table2 = word_table.reshape(V // 2, 2 * D)
    pos1d = pos_table.reshape(S * D)
    mesh = plsc.VectorSubcoreMesh(core_axis_name="c", subcore_axis_name="s")

    @functools.partial(
        pl.kernel,
        out_type=jax.ShapeDtypeStruct((B * S, D), jnp.float32),
        mesh=mesh,
        scratch_types=[
            pltpu.VMEM((SP,), jnp.int32),
            pltpu.VMEM((SP,), jnp.int32),
            pltpu.VMEM((SP, 2 * D), jnp.float32),
            pltpu.VMEM((SP, D), jnp.float32),
            pltpu.VMEM((SP * D,), jnp.float32),
            pltpu.SemaphoreType.DMA,
        ],
        compiler_params=pltpu.CompilerParams(use_tc_tiling_on_sc=True),
    )
    def emb_kernel(idx_hbm, table2_hbm, pos_hbm, out_hbm,
                   idx_v, idx2_v, rows_v, sel_v, pos_v, gsem):
        wid = lax.axis_index("s") * NC + lax.axis_index("c")
        base = wid * seqs_per_w * S
        pltpu.sync_copy(pos_hbm, pos_v.at[pl.ds(0, S * D)])
        if SP != S:
            idx_v[pl.ds(SP - L, L)] = jnp.zeros((L,), jnp.int32)

        def body(b, carry):
            start = base + b * S
            pltpu.sync_copy(idx_hbm.at[pl.ds(start, S)], idx_v.at[pl.ds(0, S)])

            def tloop(g, c2):
                sl = pl.ds(g * L, L)
                idx2_v[sl] = lax.shift_right_logical(idx_v[sl], 1)
                return c2

            lax.fori_loop(0, NG, tloop, 0)
            pltpu.async_copy(table2_hbm.at[idx2_v], rows_v, gsem).wait()

            def gloop(g, c2):
                vi = idx_v[pl.ds(g * L, L)]
                for k in range(L):
                    row = g * L + k
                    tok = lax.gather(
                        vi, jnp.full((L, 1), k, jnp.int32),
                        lax.GatherDimensionNumbers(
                            offset_dims=(), collapsed_slice_dims=(0,),
                            start_index_map=(0,)),
                        slice_sizes=(1,),
                        mode=lax.GatherScatterMode.PROMISE_IN_BOUNDS)
                    oddf = (tok & 1).astype(jnp.float32)
                    for j in range(JD):
                        lo = rows_v[row, pl.ds(j * L, L)]
                        hi = rows_v[row, pl.ds(D + j * L, L)]
                        val = lo + (hi - lo) * oddf
                        sel_v[row, pl.ds(j * L, L)] = (
                            val + pos_v[pl.ds(row * D + j * L, L)]
                        )
                return c2

            lax.fori_loop(0, NG, gloop, 0)
            pltpu.sync_copy(sel_v.at[pl.ds(0, S)], out_hbm.at[pl.ds(start, S)])
            return carry

        lax.fori_loop(0, seqs_per_w, body, 0)

    out = emb_kernel(idx_flat, table2, pos1d)
    return out.reshape(B, S, D)
